# Initial kernel scaffold; baseline (speedup 1.0000x reference)
import jax
import jax.numpy as jnp
from jax.experimental import pallas as pl


def kernel(reg, cls):
    raise NotImplementedError("write your pallas kernel here")


# speedup vs baseline: 1.8109x; 1.8109x over previous
import functools

import jax
import jax.numpy as jnp
from jax import lax
from jax.experimental import pallas as pl
from jax.experimental.pallas import tpu as pltpu
from jax.experimental.pallas import tpu_sc as plsc

B = 16
N = 20000
TOP = 100
PAD = 112
NBUCKET = 256
CHUNKS = N // 16
CAP = 4096
NEG = -3.0e38
BIG = 0x7FFFFFFF
THR = 0.5

_mesh = plsc.VectorSubcoreMesh(core_axis_name="c", subcore_axis_name="s")


@functools.partial(
    pl.kernel,
    out_type=[
        jax.ShapeDtypeStruct((B, TOP * 4), jnp.float32),
        jax.ShapeDtypeStruct((B, PAD), jnp.float32),
    ],
    mesh=_mesh,
    compiler_params=pltpu.CompilerParams(needs_layout_passes=False),
    scratch_types=[
        pltpu.VMEM((N,), jnp.float32),
        pltpu.VMEM((NBUCKET * 16,), jnp.int32),
        pltpu.VMEM((NBUCKET,), jnp.int32),
        pltpu.VMEM((CAP,), jnp.float32),
        pltpu.VMEM((CAP,), jnp.int32),
        pltpu.VMEM((PAD,), jnp.int32),
        pltpu.VMEM((PAD,), jnp.float32),
        pltpu.VMEM((PAD, 128), jnp.float32),
        pltpu.VMEM((PAD,), jnp.int32),
        pltpu.VMEM((PAD,), jnp.int32),
        pltpu.VMEM((PAD,), jnp.float32),
        pltpu.VMEM((PAD,), jnp.float32),
        pltpu.VMEM((PAD,), jnp.float32),
        pltpu.VMEM((PAD,), jnp.float32),
        pltpu.VMEM((PAD,), jnp.float32),
        pltpu.VMEM((PAD,), jnp.float32),
        pltpu.VMEM((TOP * 4,), jnp.float32),
        pltpu.VMEM((PAD,), jnp.float32),
        pltpu.SemaphoreType.DMA,
    ],
)
def _nms_sc(reg_hbm, cls_hbm, out_reg_hbm, out_cls_hbm,
            cls_v, hist, totals, cand_val, cand_idx, top_idx, top_val,
            rows_v, gidx, glo, x1s, y1s, x2s, y2s, areas, alive, orb, ocb,
            sem):
    wid = lax.axis_index("s") * 2 + lax.axis_index("c")

    @pl.when(wid < B)
    def _body():
        b = wid
        lane = jnp.arange(16, dtype=jnp.int32)
        ones_i = jnp.ones((16,), jnp.int32)
        zeros_f = jnp.zeros((16,), jnp.float32)
        neg_f = jnp.full((16,), NEG, jnp.float32)

        pltpu.sync_copy(cls_hbm.at[b], cls_v)

        def _zh(k, _):
            hist[pl.ds(k * 16, 16)] = jnp.zeros((16,), jnp.int32)
            return 0
        lax.fori_loop(0, NBUCKET, _zh, 0)

        def _bucket(v):
            bk = (v * jnp.float32(NBUCKET)).astype(jnp.int32)
            return jnp.clip(bk, 0, NBUCKET - 1)

        def _hb(c, _):
            base = c * 80
            for u in range(5):
                v = cls_v[pl.ds(base + u * 16, 16)]
                bk = _bucket(v)
                plsc.addupdate_scatter(hist, [bk * 16 + lane], ones_i)
            return 0
        lax.fori_loop(0, CHUNKS // 5, _hb, 0)

        def _tt(kc, _):
            acc = jnp.zeros((16,), jnp.int32)
            jbase = (kc * 16 + lane) * 16
            for l in range(16):
                acc = acc + plsc.load_gather(hist, [jbase + l])
            totals[pl.ds(kc * 16, 16)] = acc
            return 0
        lax.fori_loop(0, NBUCKET // 16, _tt, 0)

        def _fb(k, carry):
            cum, bsel = carry
            kk = NBUCKET - 1 - k
            t = plsc.load_gather(totals, [jnp.full((16,), kk, jnp.int32)])
            cum2 = cum + jnp.max(t)
            bsel2 = jnp.where((cum < TOP) & (cum2 >= TOP), kk, bsel)
            return (cum2, bsel2)
        _, bsel = lax.fori_loop(0, NBUCKET, _fb,
                                (jnp.int32(0), jnp.int32(0)))

        def _cp(c, off):
            v = cls_v[pl.ds(c * 16, 16)]
            m = _bucket(v) >= bsel

            def _append(o):
                cs = plsc.cumsum(m.astype(jnp.int32))
                pos = jnp.minimum(o + cs - 1, CAP - 17)
                plsc.store_scatter(cand_val, [pos], v, mask=m)
                plsc.store_scatter(cand_idx, [pos], c * 16 + lane, mask=m)
                return jnp.minimum(o + cs[15], CAP - 17)

            return lax.cond(jnp.any(m), _append, lambda o: o, off)
        m_end = lax.fori_loop(0, CHUNKS, _cp, jnp.int32(0))

        plsc.store_scatter(cand_val, [m_end + lane], neg_f)
        nch = (m_end + 16) // 16

        for c in range(PAD // 16):
            top_idx[pl.ds(c * 16, 16)] = jnp.zeros((16,), jnp.int32)
            top_val[pl.ds(c * 16, 16)] = zeros_f

        def _sel(k, _):
            def _mx(c, bvbp):
                bv, bp = bvbp
                v = cand_val[pl.ds(c * 16, 16)]
                p = c * 16 + lane
                gt = v > bv
                return (jnp.where(gt, v, bv), jnp.where(gt, p, bp))
            bv, bp = lax.fori_loop(0, nch, _mx, (neg_f, jnp.zeros((16,), jnp.int32)))
            maxv = jnp.max(bv)
            pm = jnp.where(bv == maxv, bp, BIG)
            minpos = jnp.full((16,), jnp.min(pm), jnp.int32)
            gi = plsc.load_gather(cand_idx, [minpos])
            l0 = lane == 0
            kk = jnp.full((16,), k, jnp.int32)
            plsc.store_scatter(top_idx, [kk], gi, mask=l0)
            plsc.store_scatter(top_val, [kk], jnp.full((16,), maxv, jnp.float32), mask=l0)
            plsc.store_scatter(cand_val, [minpos], neg_f, mask=l0)
            return 0
        lax.fori_loop(0, TOP, _sel, 0)

        for c in range(PAD // 16):
            sl = pl.ds(c * 16, 16)
            g = top_idx[sl] + b * N
            gidx[sl] = lax.shift_right_logical(g, 5)
            glo[sl] = (g & 31) * 4
        pltpu.async_copy(reg_hbm.at[gidx], rows_v, sem).wait()

        for c in range(PAD // 16):
            sl = pl.ds(c * 16, 16)
            j = lane + c * 16
            lo = glo[sl]
            x1 = plsc.load_gather(rows_v, [j, lo])
            y1 = plsc.load_gather(rows_v, [j, lo + 1])
            x2 = plsc.load_gather(rows_v, [j, lo + 2])
            y2 = plsc.load_gather(rows_v, [j, lo + 3])
            x1s[sl] = x1
            y1s[sl] = y1
            x2s[sl] = x2
            y2s[sl] = y2
            areas[sl] = (x2 - x1) * (y2 - y1)
            alive[sl] = jnp.where(j < TOP, 1.0, 0.0).astype(jnp.float32)

        def _nms(i, _):
            si = jnp.full((16,), i, jnp.int32)
            xi = plsc.load_gather(x1s, [si])
            yi = plsc.load_gather(y1s, [si])
            Xi = plsc.load_gather(x2s, [si])
            Yi = plsc.load_gather(y2s, [si])
            ai = plsc.load_gather(areas, [si])
            live_i = plsc.load_gather(alive, [si]) > 0.5
            for c in range(PAD // 16):
                sl = pl.ds(c * 16, 16)
                w = jnp.maximum(jnp.minimum(Xi, x2s[sl]) - jnp.maximum(xi, x1s[sl]), 0.0)
                h = jnp.maximum(jnp.minimum(Yi, y2s[sl]) - jnp.maximum(yi, y1s[sl]), 0.0)
                inter = w * h
                un = jnp.maximum(ai + areas[sl] - inter, 1e-8)
                keep = (inter < THR * un) | (lane + c * 16 == si)
                ac = alive[sl]
                alive[sl] = jnp.where(live_i, jnp.where(keep, ac, 0.0), ac)
            return 0
        lax.fori_loop(0, TOP, _nms, 0)

        for c in range(PAD // 16):
            sl = pl.ds(c * 16, 16)
            ocb[sl] = top_val[sl] * alive[sl]
        for c in range(TOP * 4 // 16):
            sl = pl.ds(c * 16, 16)
            p = lane + c * 16
            j = lax.shift_right_logical(p, 2)
            d = p & 3
            lo = plsc.load_gather(glo, [j])
            v = plsc.load_gather(rows_v, [j, lo + d])
            a = plsc.load_gather(alive, [j])
            orb[sl] = v * a
        pltpu.sync_copy(orb, out_reg_hbm.at[b])
        pltpu.sync_copy(ocb, out_cls_hbm.at[b])


def kernel(reg, cls):
    reg2 = reg.reshape(B * N * 4 // 128, 128)
    out_reg_flat, out_cls_pad = _nms_sc(reg2, cls)
    return out_reg_flat.reshape(B, TOP, 4), out_cls_pad[:, :TOP]


# speedup vs baseline: 3.5390x; 1.9543x over previous
import functools

import jax
import jax.numpy as jnp
from jax import lax
from jax.experimental import pallas as pl
from jax.experimental.pallas import tpu as pltpu
from jax.experimental.pallas import tpu_sc as plsc

B = 16
N = 20000
TOP = 100
PAD = 112
NBUCKET = 256
CHUNKS = N // 16
CAP = 4096
NEG = -3.0e38
BIG = 0x7FFFFFFF
THR = 0.5

_mesh = plsc.VectorSubcoreMesh(core_axis_name="c", subcore_axis_name="s")


@functools.partial(
    pl.kernel,
    out_type=[
        jax.ShapeDtypeStruct((B, PAD), jnp.int32),
        jax.ShapeDtypeStruct((B, PAD), jnp.float32),
    ],
    mesh=_mesh,
    compiler_params=pltpu.CompilerParams(needs_layout_passes=False),
    scratch_types=[
        pltpu.VMEM((N,), jnp.float32),
        pltpu.VMEM((NBUCKET * 16,), jnp.int32),
        pltpu.VMEM((NBUCKET,), jnp.int32),
        pltpu.VMEM((CAP,), jnp.float32),
        pltpu.VMEM((CAP,), jnp.int32),
        pltpu.VMEM((PAD,), jnp.int32),
        pltpu.VMEM((PAD,), jnp.float32),
    ],
)
def _topk_sc(cls_hbm, idx_hbm, val_hbm,
             cls_v, hist, totals, cand_val, cand_idx, top_idx, top_val):
    wid = lax.axis_index("s") * 2 + lax.axis_index("c")

    @pl.when(wid < B)
    def _body():
        b = wid
        lane = jnp.arange(16, dtype=jnp.int32)
        ones_i = jnp.ones((16,), jnp.int32)
        zeros_f = jnp.zeros((16,), jnp.float32)
        neg_f = jnp.full((16,), NEG, jnp.float32)

        pltpu.sync_copy(cls_hbm.at[b], cls_v)

        def _zh(k, _):
            hist[pl.ds(k * 16, 16)] = jnp.zeros((16,), jnp.int32)
            return 0
        lax.fori_loop(0, NBUCKET, _zh, 0)

        def _bucket(v):
            bk = (v * jnp.float32(NBUCKET)).astype(jnp.int32)
            return jnp.clip(bk, 0, NBUCKET - 1)

        def _hb(c, _):
            base = c * 80
            for u in range(5):
                v = cls_v[pl.ds(base + u * 16, 16)]
                bk = _bucket(v)
                plsc.addupdate_scatter(hist, [bk * 16 + lane], ones_i)
            return 0
        lax.fori_loop(0, CHUNKS // 5, _hb, 0)

        def _tt(kc, _):
            acc = jnp.zeros((16,), jnp.int32)
            jbase = (kc * 16 + lane) * 16
            for l in range(16):
                acc = acc + plsc.load_gather(hist, [jbase + l])
            totals[pl.ds(kc * 16, 16)] = acc
            return 0
        lax.fori_loop(0, NBUCKET // 16, _tt, 0)

        def _fb(k, carry):
            cum, bsel = carry
            kk = NBUCKET - 1 - k
            t = plsc.load_gather(totals, [jnp.full((16,), kk, jnp.int32)])
            cum2 = cum + jnp.max(t)
            bsel2 = jnp.where((cum < TOP) & (cum2 >= TOP), kk, bsel)
            return (cum2, bsel2)
        _, bsel = lax.fori_loop(0, NBUCKET, _fb,
                                (jnp.int32(0), jnp.int32(0)))

        def _cp(c, off):
            v = cls_v[pl.ds(c * 16, 16)]
            m = _bucket(v) >= bsel

            def _append(o):
                cs = plsc.cumsum(m.astype(jnp.int32))
                pos = jnp.minimum(o + cs - 1, CAP - 17)
                plsc.store_scatter(cand_val, [pos], v, mask=m)
                plsc.store_scatter(cand_idx, [pos], c * 16 + lane, mask=m)
                return jnp.minimum(o + cs[15], CAP - 17)

            return lax.cond(jnp.any(m), _append, lambda o: o, off)
        m_end = lax.fori_loop(0, CHUNKS, _cp, jnp.int32(0))

        plsc.store_scatter(cand_val, [m_end + lane], neg_f)
        nch = (m_end + 16) // 16

        for c in range(PAD // 16):
            top_idx[pl.ds(c * 16, 16)] = jnp.zeros((16,), jnp.int32)
            top_val[pl.ds(c * 16, 16)] = zeros_f

        def _sel(k, _):
            def _mx(c, bvbp):
                bv, bp = bvbp
                v = cand_val[pl.ds(c * 16, 16)]
                p = c * 16 + lane
                gt = v > bv
                return (jnp.where(gt, v, bv), jnp.where(gt, p, bp))
            bv, bp = lax.fori_loop(0, nch, _mx,
                                   (neg_f, jnp.zeros((16,), jnp.int32)))
            maxv = jnp.max(bv)
            pm = jnp.where(bv == maxv, bp, BIG)
            minpos = jnp.full((16,), jnp.min(pm), jnp.int32)
            gi = plsc.load_gather(cand_idx, [minpos])
            l0 = lane == 0
            kk = jnp.full((16,), k, jnp.int32)
            plsc.store_scatter(top_idx, [kk], gi, mask=l0)
            plsc.store_scatter(top_val, [kk],
                               jnp.full((16,), maxv, jnp.float32), mask=l0)
            plsc.store_scatter(cand_val, [minpos], neg_f, mask=l0)
            return 0
        lax.fori_loop(0, TOP, _sel, 0)

        pltpu.sync_copy(top_idx, idx_hbm.at[b])
        pltpu.sync_copy(top_val, val_hbm.at[b])


def _gather_body(idx_sref, r_ref, o_ref):
    b = pl.program_id(0)
    for j in range(PAD):
        ij = idx_sref[b * PAD + j]
        o_ref[0, j, pl.ds(0, 4)] = r_ref[0, pl.ds(ij, 1), :][0]


_gather_tc = pl.pallas_call(
    _gather_body,
    grid_spec=pltpu.PrefetchScalarGridSpec(
        num_scalar_prefetch=1,
        grid=(B,),
        in_specs=[pl.BlockSpec((1, N, 4), lambda b, *_: (b, 0, 0))],
        out_specs=pl.BlockSpec((1, PAD, 128), lambda b, *_: (b, 0, 0)),
    ),
    out_shape=jax.ShapeDtypeStruct((B, PAD, 128), jnp.float32),
)


@functools.partial(
    pl.kernel,
    out_type=[
        jax.ShapeDtypeStruct((B, TOP * 4), jnp.float32),
        jax.ShapeDtypeStruct((B, PAD), jnp.float32),
    ],
    mesh=_mesh,
    compiler_params=pltpu.CompilerParams(needs_layout_passes=False),
    scratch_types=[
        pltpu.VMEM((PAD, 128), jnp.float32),
        pltpu.VMEM((PAD,), jnp.float32),
        pltpu.VMEM((PAD,), jnp.float32),
        pltpu.VMEM((PAD,), jnp.float32),
        pltpu.VMEM((PAD,), jnp.float32),
        pltpu.VMEM((PAD,), jnp.float32),
        pltpu.VMEM((PAD,), jnp.float32),
        pltpu.VMEM((PAD,), jnp.float32),
        pltpu.VMEM((TOP * 4,), jnp.float32),
        pltpu.VMEM((PAD,), jnp.float32),
    ],
)
def _nms_sc(rows_hbm, val_hbm, out_reg_hbm, out_cls_hbm,
            rows_v, tval, x1s, y1s, x2s, y2s, areas, alive, orb, ocb):
    wid = lax.axis_index("s") * 2 + lax.axis_index("c")

    @pl.when(wid < B)
    def _body():
        b = wid
        lane = jnp.arange(16, dtype=jnp.int32)

        pltpu.sync_copy(rows_hbm.at[b], rows_v)
        pltpu.sync_copy(val_hbm.at[b], tval)

        zi = jnp.zeros((16,), jnp.int32)
        for c in range(PAD // 16):
            sl = pl.ds(c * 16, 16)
            j = lane + c * 16
            x1 = plsc.load_gather(rows_v, [j, zi])
            y1 = plsc.load_gather(rows_v, [j, zi + 1])
            x2 = plsc.load_gather(rows_v, [j, zi + 2])
            y2 = plsc.load_gather(rows_v, [j, zi + 3])
            x1s[sl] = x1
            y1s[sl] = y1
            x2s[sl] = x2
            y2s[sl] = y2
            areas[sl] = (x2 - x1) * (y2 - y1)
            alive[sl] = jnp.where(j < TOP, 1.0, 0.0).astype(jnp.float32)

        def _nms(i, _):
            si = jnp.full((16,), i, jnp.int32)
            xi = plsc.load_gather(x1s, [si])
            yi = plsc.load_gather(y1s, [si])
            Xi = plsc.load_gather(x2s, [si])
            Yi = plsc.load_gather(y2s, [si])
            ai = plsc.load_gather(areas, [si])
            live_i = plsc.load_gather(alive, [si]) > 0.5
            for c in range(PAD // 16):
                sl = pl.ds(c * 16, 16)
                w = jnp.maximum(
                    jnp.minimum(Xi, x2s[sl]) - jnp.maximum(xi, x1s[sl]), 0.0)
                h = jnp.maximum(
                    jnp.minimum(Yi, y2s[sl]) - jnp.maximum(yi, y1s[sl]), 0.0)
                inter = w * h
                un = jnp.maximum(ai + areas[sl] - inter, 1e-8)
                keep = (inter < THR * un) | (lane + c * 16 == si)
                ac = alive[sl]
                alive[sl] = jnp.where(live_i, jnp.where(keep, ac, 0.0), ac)
            return 0
        lax.fori_loop(0, TOP, _nms, 0)

        for c in range(PAD // 16):
            sl = pl.ds(c * 16, 16)
            ocb[sl] = tval[sl] * alive[sl]
        for c in range(TOP * 4 // 16):
            sl = pl.ds(c * 16, 16)
            p = lane + c * 16
            j = lax.shift_right_logical(p, 2)
            d = p & 3
            v = plsc.load_gather(rows_v, [j, d])
            a = plsc.load_gather(alive, [j])
            orb[sl] = v * a
        pltpu.sync_copy(orb, out_reg_hbm.at[b])
        pltpu.sync_copy(ocb, out_cls_hbm.at[b])


def kernel(reg, cls):
    top_idx, top_val = _topk_sc(cls)
    rows = _gather_tc(top_idx.reshape(-1), reg)
    out_reg_flat, out_cls_pad = _nms_sc(rows, top_val)
    return out_reg_flat.reshape(B, TOP, 4), out_cls_pad[:, :TOP]
